# baseline (speedup 1.0000x reference)
import functools

import jax
import jax.numpy as jnp
from jax.experimental import pallas as pl
from jax.experimental.pallas import tpu as pltpu

_EB = 4000
_RC = 1000


def _edge_loop(src_ref, dst_ref, pos_ref, y_ref, h_scr, w1v, cnt_ref):
    def body(j, _):
        s = src_ref[0, 0, j]
        d = dst_ref[0, 0, j]
        ps = pos_ref[pl.ds(s, 1), :]
        pd = pos_ref[pl.ds(d, 1), :]
        diff = pd - ps
        ef = jnp.sqrt(jnp.sum(diff * diff, axis=1, keepdims=True))
        row = y_ref[pl.ds(d, 1), :]
        contrib = jnp.maximum(row + ef * w1v, 0.0)
        h_scr[pl.ds(d, 1), :] += contrib
        if cnt_ref is not None:
            cnt_ref[pl.ds(d, 1), :] += 1.0
        return 0

    jax.lax.fori_loop(0, _EB, body, 0)


def _init_chunked(x_ref, w1a_ref, b1_ref, x_out, h_scr, cnt_ref, n):
    w1a = w1a_ref[:]
    b1v = b1_ref[:]
    for c in range(0, n, _RC):
        sl = pl.ds(c, _RC)
        x_out[sl, :] = jnp.dot(x_ref[sl, :], w1a,
                               preferred_element_type=jnp.float32) + b1v
        h_scr[sl, :] = jnp.zeros((_RC, h_scr.shape[1]), jnp.float32)
        if cnt_ref is not None:
            cnt_ref[sl, :] = jnp.zeros((_RC, cnt_ref.shape[1]), jnp.float32)


def _finish_chunked(h_scr, w2_ref, b2_ref, wc_ref, bc_ref, pos_ref, v_ref,
                    x_out, pos_out, n, layer0):
    w2 = w2_ref[:]
    b2v = b2_ref[:]
    wcv = wc_ref[:]
    bcs = bc_ref[0, 0]
    for c in range(0, n, _RC):
        sl = pl.ds(c, _RC)
        if layer0:
            cnt = v_ref[sl, :]
            vc = jnp.where(cnt > 0.0, 1.0 / jnp.maximum(cnt, 1.0), 0.0)
            v_ref[sl, :] = vc
        else:
            vc = v_ref[sl, :]
        iv = vc[:, 0:1]
        cz = jnp.where(iv > 0.0, 1.0, 0.0)
        aggr = jnp.dot(h_scr[sl, :], w2,
                       preferred_element_type=jnp.float32) * iv + cz * b2v
        x_out[sl, :] = aggr
        cp = jnp.sum(aggr * wcv, axis=1, keepdims=True) + bcs
        pos_out[sl, :] = pos_ref[sl, :] + jnp.tanh(cp)


def _layer_kernel(src_ref, dst_ref, x_ref, pos_ref, w1a_ref, b1_ref, w1v_ref,
                  w2_ref, b2_ref, wc_ref, bc_ref, v_ref,
                  x_out, pos_out, h_scr, *, nb, n):
    pid = pl.program_id(0)

    @pl.when(pid == 0)
    def _init():
        _init_chunked(x_ref, w1a_ref, b1_ref, x_out, h_scr, None, n)

    _edge_loop(src_ref, dst_ref, pos_ref, x_out, h_scr, w1v_ref[:], None)

    @pl.when(pid == nb - 1)
    def _final():
        _finish_chunked(h_scr, w2_ref, b2_ref, wc_ref, bc_ref, pos_ref, v_ref,
                        x_out, pos_out, n, layer0=False)


def _layer0_kernel(src_ref, dst_ref, x_ref, pos_ref, w1a_ref, b1_ref, w1v_ref,
                   w2_ref, b2_ref, wc_ref, bc_ref,
                   x_out, pos_out, v_out, h_scr, *, nb, n):
    pid = pl.program_id(0)

    @pl.when(pid == 0)
    def _init():
        _init_chunked(x_ref, w1a_ref, b1_ref, x_out, h_scr, v_out, n)

    _edge_loop(src_ref, dst_ref, pos_ref, x_out, h_scr, w1v_ref[:], v_out)

    @pl.when(pid == nb - 1)
    def _final():
        _finish_chunked(h_scr, w2_ref, b2_ref, wc_ref, bc_ref, pos_ref, v_out,
                        x_out, pos_out, n, layer0=True)


@jax.jit
def kernel(x, pos, edge_index, W1, b1, W2, b2, Wc, bc):
    n, d = x.shape
    e = edge_index.shape[1]
    num_layers = W1.shape[0]
    nb = e // _EB

    src = edge_index[0].reshape(nb, 1, _EB)
    dst = edge_index[1].reshape(nb, 1, _EB)

    whole = lambda a: pl.BlockSpec(a.shape, lambda i: (0,) * a.ndim)
    smem_edges = pl.BlockSpec((1, 1, _EB), lambda i: (i, 0, 0),
                              memory_space=pltpu.SMEM)
    bc_spec = pl.BlockSpec((1, 1), lambda i: (0, 0), memory_space=pltpu.SMEM)

    v = None
    for l in range(num_layers):
        w1a = W1[l, :d, :]
        w1v = W1[l, d, :].reshape(1, d)
        b1l = b1[l].reshape(1, d)
        w2l = W2[l]
        b2l = b2[l].reshape(1, d)
        wcl = Wc[l].reshape(1, d)
        bcl = bc[l].reshape(1, 1)

        common_in = [x, pos, w1a, b1l, w1v, w2l, b2l, wcl]
        common_specs = [whole(a) for a in common_in]

        if l == 0:
            x, pos, v = pl.pallas_call(
                functools.partial(_layer0_kernel, nb=nb, n=n),
                grid=(nb,),
                in_specs=[smem_edges, smem_edges] + common_specs + [bc_spec],
                out_specs=(whole(x), whole(pos),
                           pl.BlockSpec((n, 128), lambda i: (0, 0))),
                out_shape=(jax.ShapeDtypeStruct((n, d), jnp.float32),
                           jax.ShapeDtypeStruct((n, 3), jnp.float32),
                           jax.ShapeDtypeStruct((n, 128), jnp.float32)),
                scratch_shapes=[pltpu.VMEM((n, d), jnp.float32)],
            )(src, dst, *common_in, bcl)
        else:
            x, pos = pl.pallas_call(
                functools.partial(_layer_kernel, nb=nb, n=n),
                grid=(nb,),
                in_specs=([smem_edges, smem_edges] + common_specs
                          + [bc_spec, whole(v)]),
                out_specs=(whole(x), whole(pos)),
                out_shape=(jax.ShapeDtypeStruct((n, d), jnp.float32),
                           jax.ShapeDtypeStruct((n, 3), jnp.float32)),
                scratch_shapes=[pltpu.VMEM((n, d), jnp.float32)],
            )(src, dst, *common_in, bcl, v)

    return (x, pos)
